# baseline (speedup 1.0000x reference)
import functools

import jax
import jax.numpy as jnp
from jax import lax
from jax.experimental import pallas as pl
from jax.experimental.pallas import tpu as pltpu
from jax.experimental.pallas import tpu_sc as plsc

_BATCH = 16384
_EMB = 64

_info = plsc.get_sparse_core_info()
_NC = _info.num_cores
_NS = _info.num_subcores
_NW = _NC * _NS
_BPW = _BATCH // _NW
_CH = 128
_NCH = _BPW // _CH

_sc_mesh = plsc.VectorSubcoreMesh(core_axis_name="c", subcore_axis_name="s")


@functools.partial(
    pl.kernel,
    out_type=(
        jax.ShapeDtypeStruct((_BATCH, _EMB), jnp.float32),
        jax.ShapeDtypeStruct((_BATCH, _EMB), jnp.float32),
    ),
    mesh=_sc_mesh,
    scratch_types=[
        pltpu.VMEM((_BPW,), jnp.int32),
        pltpu.VMEM((_BPW,), jnp.int32),
        pltpu.VMEM((_BPW, _EMB), jnp.float32),
        pltpu.VMEM((_BPW, _EMB), jnp.float32),
        pltpu.SemaphoreType.DMA,
    ],
    compiler_params=pltpu.CompilerParams(use_tc_tiling_on_sc=False),
)
def _sc_gather(uid_hbm, iid_hbm, utab_hbm, itab_hbm, u_out, i_out,
               uidx, iidx, urows, irows, sem):
    wid = lax.axis_index("s") * _NC + lax.axis_index("c")
    base = wid * _BPW
    pltpu.sync_copy(uid_hbm.at[pl.ds(base, _BPW)], uidx)
    pltpu.sync_copy(iid_hbm.at[pl.ds(base, _BPW)], iidx)
    copies = []
    for j in range(_NCH):
        sl = pl.ds(j * _CH, _CH)
        copies.append(pltpu.async_copy(utab_hbm.at[uidx.at[sl]], urows.at[sl], sem))
        copies.append(pltpu.async_copy(itab_hbm.at[iidx.at[sl]], irows.at[sl], sem))
    for c in copies:
        c.wait()
    pltpu.sync_copy(urows, u_out.at[pl.ds(base, _BPW)])
    pltpu.sync_copy(irows, i_out.at[pl.ds(base, _BPW)])


_ROWS = 2048


def _mlp_body(u_ref, i_ref, wlin_ref, w1_ref, w2_ref, w3_ref,
              b1_ref, b2_ref, scal_ref, o_ref):
    u = u_ref[...]
    it = i_ref[...]
    w1 = w1_ref[...]
    h1 = jnp.dot(u, w1[:_EMB], preferred_element_type=jnp.float32)
    h1 = h1 + jnp.dot(it, w1[_EMB:], preferred_element_type=jnp.float32)
    h1 = jnp.maximum(h1 + b1_ref[...], 0.0)
    h2 = jnp.dot(h1, w2_ref[...], preferred_element_type=jnp.float32)
    h2 = jnp.maximum(h2 + b2_ref[...], 0.0)
    neural = jnp.dot(h2, w3_ref[...], preferred_element_type=jnp.float32)
    wlin = wlin_ref[...]
    linear = jnp.dot(u, wlin[:_EMB], preferred_element_type=jnp.float32)
    linear = linear + jnp.dot(it, wlin[_EMB:], preferred_element_type=jnp.float32)
    b_lin = scal_ref[0, 0]
    b3 = scal_ref[0, 1]
    g = jax.nn.sigmoid(scal_ref[0, 2])
    o_ref[...] = g * (linear + b_lin) + (1.0 - g) * (neural + b3)


def _mlp_call(u_emb, i_emb, W_lin, W1, W2, W3, b1, b2, scalars):
    n_blocks = _BATCH // _ROWS
    full = lambda shape: pl.BlockSpec(shape, lambda i: (0,) * len(shape))
    return pl.pallas_call(
        _mlp_body,
        grid=(n_blocks,),
        in_specs=[
            pl.BlockSpec((_ROWS, _EMB), lambda i: (i, 0)),
            pl.BlockSpec((_ROWS, _EMB), lambda i: (i, 0)),
            full((2 * _EMB, 1)),
            full((2 * _EMB, _EMB)),
            full((_EMB, 32)),
            full((32, 1)),
            full((1, _EMB)),
            full((1, 32)),
            full((1, 3)),
        ],
        out_specs=pl.BlockSpec((_ROWS, 1), lambda i: (i, 0)),
        out_shape=jax.ShapeDtypeStruct((_BATCH, 1), jnp.float32),
        compiler_params=pltpu.CompilerParams(
            dimension_semantics=("arbitrary",),
        ),
    )(u_emb, i_emb, W_lin, W1, W2, W3, b1, b2, scalars)


def kernel(user_ids, item_ids, user_table, item_table,
           W_lin, b_lin, W1, b1, W2, b2, W3, b3, alpha):
    u_emb, i_emb = _sc_gather(user_ids, item_ids, user_table, item_table)
    scalars = jnp.stack([b_lin[0], b3[0], alpha[0]]).reshape(1, 3)
    return _mlp_call(u_emb, i_emb, W_lin, W1, W2, W3,
                     b1.reshape(1, _EMB), b2.reshape(1, 32), scalars)
